# baseline (speedup 1.0000x reference)
import jax
import jax.numpy as jnp
from jax import lax
from jax.experimental import pallas as pl
from jax.experimental.pallas import tpu as pltpu
from jax.experimental.pallas import tpu_sc as plsc

_ROWS = 2
_COLS = 128


def _scs_body(x_hbm, y_hbm, o_hbm, xs, ys, sem):
    c = lax.axis_index("c")

    @pl.when(c == 0)
    def _():
        cpx = pltpu.async_copy(x_hbm.at[pl.ds(0, _ROWS)], xs, sem)
        cpy = pltpu.async_copy(y_hbm.at[pl.ds(0, _ROWS)], ys, sem)
        cpx.wait()
        cpy.wait()
        for i in range(_ROWS):
            for j in range(_COLS):
                xs[i, j] = xs[i, j] + ys[i, j]
        pltpu.sync_copy(xs, o_hbm)


def kernel(x, y):
    f = pl.kernel(
        _scs_body,
        out_type=jax.ShapeDtypeStruct((_ROWS, _COLS), jnp.float32),
        mesh=plsc.ScalarSubcoreMesh(axis_name="c", num_cores=1),
        scratch_types=[
            pltpu.SMEM((_ROWS, _COLS), jnp.float32),
            pltpu.SMEM((_ROWS, _COLS), jnp.float32),
            pltpu.SemaphoreType.DMA,
        ],
    )
    return f(x, y)


# speedup vs baseline: 1.0135x; 1.0135x over previous
import jax
import jax.numpy as jnp
from jax import lax
from jax.experimental import pallas as pl
from jax.experimental.pallas import tpu as pltpu
from jax.experimental.pallas import tpu_sc as plsc

_ROWS = 2
_COLS = 128
_LANES = 16


def _sc_gather_add(x_hbm, y_hbm, o_hbm, xv, yv, sem):
    c = lax.axis_index("c")
    s = lax.axis_index("s")

    @pl.when(jnp.logical_and(c == 0, s == 0))
    def _():
        cpx = pltpu.async_copy(x_hbm.at[pl.ds(0, _ROWS)], xv, sem)
        cpy = pltpu.async_copy(y_hbm.at[pl.ds(0, _ROWS)], yv, sem)
        cpx.wait()
        cpy.wait()
        for i in range(_ROWS):
            for j in range(_COLS // _LANES):
                sl = pl.ds(j * _LANES, _LANES)
                xv[i, sl] = xv[i, sl] + yv[i, sl]
        pltpu.sync_copy(xv, o_hbm)


def kernel(x, y):
    f = pl.kernel(
        _sc_gather_add,
        out_type=jax.ShapeDtypeStruct((_ROWS, _COLS), jnp.float32),
        mesh=plsc.VectorSubcoreMesh(
            core_axis_name="c", subcore_axis_name="s", num_cores=1
        ),
        scratch_types=[
            pltpu.VMEM((_ROWS, _COLS), jnp.float32),
            pltpu.VMEM((_ROWS, _COLS), jnp.float32),
            pltpu.SemaphoreType.DMA,
        ],
    )
    return f(x, y)
